# baseline (speedup 1.0000x reference)
import functools

import jax
import jax.numpy as jnp
from jax import lax
from jax.experimental import pallas as pl
from jax.experimental.pallas import tpu as pltpu
from jax.experimental.pallas import tpu_sc as plsc

L = 16
NC = 2
NS = 16
NW = NC * NS
CH = 128


NBUF = 4


def _sc_embed(table, idx2d, scale_v, *, chunks_per_w, d):
    n = NW * chunks_per_w * CH
    assert chunks_per_w % NBUF == 0
    mesh = plsc.VectorSubcoreMesh(core_axis_name="c", subcore_axis_name="s")

    @functools.partial(
        pl.kernel,
        out_type=jax.ShapeDtypeStruct((n, d), jnp.float32),
        mesh=mesh,
        scratch_types=[
            pltpu.VMEM((chunks_per_w, CH), jnp.int32),
            pltpu.VMEM((NBUF, CH, d), jnp.float32),
            pltpu.VMEM((NBUF, CH, d), jnp.float32),
            pltpu.VMEM((L,), jnp.float32),
            pltpu.SemaphoreType.DMA((NBUF,)),
            pltpu.SemaphoreType.DMA((NBUF,)),
        ],
        compiler_params=pltpu.CompilerParams(use_tc_tiling_on_sc=False),
    )
    def body(table_hbm, idx_hbm, scale_hbm, out_hbm,
             idx_v, gbuf, sbuf, scale_sp, gsem, ssem):
        wid = lax.axis_index("s") * NC + lax.axis_index("c")
        pltpu.sync_copy(scale_hbm, scale_sp)
        pltpu.sync_copy(idx_hbm.at[pl.ds(wid * chunks_per_w, chunks_per_w), :],
                        idx_v)
        sv = scale_sp[...]
        base = wid * chunks_per_w * CH

        def fire_gather(j, b):
            pltpu.async_copy(table_hbm.at[idx_v.at[j]], gbuf.at[b], gsem.at[b])

        for b in range(NBUF):
            fire_gather(b, b)

        @pl.loop(0, chunks_per_w, step=NBUF)
        def _grp(j0):
            for b in range(NBUF):
                j = j0 + b
                pltpu.make_async_copy(table_hbm.at[idx_v.at[j]],
                                      gbuf.at[b], gsem.at[b]).wait()

                @pl.when(j0 > 0)
                def _():
                    pltpu.make_async_copy(sbuf.at[b],
                                          out_hbm.at[pl.ds(base, CH), :],
                                          ssem.at[b]).wait()

                @pl.loop(0, CH, unroll=8)
                def _row(r):
                    for c in range(d // L):
                        sl = pl.ds(c * L, L)
                        sbuf[b, r, sl] = gbuf[b, r, sl] * sv

                @pl.when(j + NBUF < chunks_per_w)
                def _():
                    fire_gather(j + NBUF, b)

                pltpu.async_copy(sbuf.at[b],
                                 out_hbm.at[pl.ds(base + j * CH, CH), :],
                                 ssem.at[b])

        for b in range(NBUF):
            pltpu.make_async_copy(sbuf.at[b], out_hbm.at[pl.ds(base, CH), :],
                                  ssem.at[b]).wait()

    return body(table, idx2d, scale_v)


def kernel(x, table, scale):
    b, h = x.shape
    v, d = table.shape
    n = b * h
    assert n % (NW * CH) == 0 and d % L == 0
    chunks_per_w = n // (NW * CH)
    idx2d = x.reshape(NW * chunks_per_w, CH).astype(jnp.int32)
    scale_v = jnp.broadcast_to(scale.astype(jnp.float32), (L,))
    out = _sc_embed(table, idx2d, scale_v, chunks_per_w=chunks_per_w, d=d)
    return out.reshape(b, h, d)


# speedup vs baseline: 1.0991x; 1.0991x over previous
import functools

import jax
import jax.numpy as jnp
from jax import lax
from jax.experimental import pallas as pl
from jax.experimental.pallas import tpu as pltpu
from jax.experimental.pallas import tpu_sc as plsc

L = 16
NC = 2
NS = 16
NW = NC * NS
CH = 128


NBUF = 4


def _sc_embed(table, idx2d, scale_v, *, chunks_per_w, d):
    n = NW * chunks_per_w * CH
    assert chunks_per_w % NBUF == 0
    mesh = plsc.VectorSubcoreMesh(core_axis_name="c", subcore_axis_name="s")

    @functools.partial(
        pl.kernel,
        out_type=jax.ShapeDtypeStruct((n, d), jnp.float32),
        mesh=mesh,
        scratch_types=[
            pltpu.VMEM((chunks_per_w, CH), jnp.int32),
            pltpu.VMEM((NBUF, CH, d), jnp.float32),
            pltpu.VMEM((NBUF, CH, d), jnp.float32),
            pltpu.VMEM((L,), jnp.float32),
            pltpu.SemaphoreType.DMA((NBUF,)),
            pltpu.SemaphoreType.DMA((NBUF,)),
        ],
        compiler_params=pltpu.CompilerParams(use_tc_tiling_on_sc=False),
    )
    def body(table_hbm, idx_hbm, scale_hbm, out_hbm,
             idx_v, gbuf, sbuf, scale_sp, gsem, ssem):
        wid = lax.axis_index("s") * NC + lax.axis_index("c")
        pltpu.sync_copy(scale_hbm, scale_sp)
        pltpu.sync_copy(idx_hbm.at[pl.ds(wid * chunks_per_w, chunks_per_w), :],
                        idx_v)
        sv = scale_sp[...]
        base = wid * chunks_per_w * CH

        def fire_gather(j, b):
            pltpu.async_copy(table_hbm.at[idx_v.at[j]], gbuf.at[b], gsem.at[b])

        for b in range(NBUF):
            fire_gather(b, b)

        @pl.loop(0, chunks_per_w, step=NBUF)
        def _grp(j0):
            for b in range(NBUF):
                j = j0 + b
                pltpu.make_async_copy(table_hbm.at[idx_v.at[j]],
                                      gbuf.at[b], gsem.at[b]).wait()

                @pl.when(j0 > 0)
                def _():
                    pltpu.make_async_copy(sbuf.at[b],
                                          out_hbm.at[pl.ds(base, CH), :],
                                          ssem.at[b]).wait()

                @pl.loop(0, CH)
                def _row(r):
                    for c in range(d // L):
                        sl = pl.ds(c * L, L)
                        sbuf[b, r, sl] = gbuf[b, r, sl] * sv

                @pl.when(j + NBUF < chunks_per_w)
                def _():
                    fire_gather(j + NBUF, b)

                pltpu.async_copy(sbuf.at[b],
                                 out_hbm.at[pl.ds(base + j * CH, CH), :],
                                 ssem.at[b])

        for b in range(NBUF):
            pltpu.make_async_copy(sbuf.at[b], out_hbm.at[pl.ds(base, CH), :],
                                  ssem.at[b]).wait()

    return body(table, idx2d, scale_v)


def kernel(x, table, scale):
    b, h = x.shape
    v, d = table.shape
    n = b * h
    assert n % (NW * CH) == 0 and d % L == 0
    chunks_per_w = n // (NW * CH)
    idx2d = x.reshape(NW * chunks_per_w, CH).astype(jnp.int32)
    scale_v = jnp.broadcast_to(scale.astype(jnp.float32), (L,))
    out = _sc_embed(table, idx2d, scale_v, chunks_per_w=chunks_per_w, d=d)
    return out.reshape(b, h, d)


# speedup vs baseline: 1.1041x; 1.0046x over previous
import functools

import jax
import jax.numpy as jnp
from jax import lax
from jax.experimental import pallas as pl
from jax.experimental.pallas import tpu as pltpu
from jax.experimental.pallas import tpu_sc as plsc

L = 16
NC = 2
NS = 16
NW = NC * NS
RPB = 2
NBUF = 4


def _sc_embed(table, idx2d, scale_v, *, rows_per_w, h, d):
    btot = rows_per_w * NW
    ch = RPB * h
    chunks_per_w = rows_per_w // RPB
    assert chunks_per_w % NBUF == 0
    mesh = plsc.VectorSubcoreMesh(core_axis_name="c", subcore_axis_name="s")

    @functools.partial(
        pl.kernel,
        out_type=jax.ShapeDtypeStruct((btot, h, d), jnp.float32),
        mesh=mesh,
        scratch_types=[
            pltpu.VMEM((chunks_per_w, ch), jnp.int32),
            pltpu.VMEM((NBUF, ch, d), jnp.float32),
            pltpu.VMEM((NBUF, RPB, h, d), jnp.float32),
            pltpu.VMEM((L,), jnp.float32),
            pltpu.SemaphoreType.DMA((NBUF,)),
            pltpu.SemaphoreType.DMA((NBUF,)),
        ],
        compiler_params=pltpu.CompilerParams(use_tc_tiling_on_sc=False),
    )
    def body(table_hbm, idx_hbm, scale_hbm, out_hbm,
             idx_v, gbuf, sbuf, scale_sp, gsem, ssem):
        wid = lax.axis_index("s") * NC + lax.axis_index("c")
        pltpu.sync_copy(scale_hbm, scale_sp)
        pltpu.sync_copy(idx_hbm.at[pl.ds(wid * chunks_per_w, chunks_per_w), :],
                        idx_v)
        sv = scale_sp[...]
        base = wid * rows_per_w

        def fire_gather(j, b):
            pltpu.async_copy(table_hbm.at[idx_v.at[j]], gbuf.at[b], gsem.at[b])

        for b in range(NBUF):
            fire_gather(b, b)

        @pl.loop(0, chunks_per_w, step=NBUF)
        def _grp(j0):
            for b in range(NBUF):
                j = j0 + b
                pltpu.make_async_copy(table_hbm.at[idx_v.at[j]],
                                      gbuf.at[b], gsem.at[b]).wait()

                @pl.when(j0 > 0)
                def _():
                    pltpu.make_async_copy(sbuf.at[b],
                                          out_hbm.at[pl.ds(base, RPB), :, :],
                                          ssem.at[b]).wait()

                for b2 in range(RPB):
                    @pl.loop(0, h)
                    def _row(r):
                        for c in range(d // L):
                            sl = pl.ds(c * L, L)
                            sbuf[b, b2, r, sl] = gbuf[b, b2 * h + r, sl] * sv

                @pl.when(j + NBUF < chunks_per_w)
                def _():
                    fire_gather(j + NBUF, b)

                pltpu.async_copy(sbuf.at[b],
                                 out_hbm.at[pl.ds(base + j * RPB, RPB), :, :],
                                 ssem.at[b])

        for b in range(NBUF):
            pltpu.make_async_copy(sbuf.at[b],
                                  out_hbm.at[pl.ds(base, RPB), :, :],
                                  ssem.at[b]).wait()

    return body(table, idx2d, scale_v)


def kernel(x, table, scale):
    b, h = x.shape
    v, d = table.shape
    assert b % (NW * RPB * NBUF) == 0 and d % L == 0 and RPB * h <= 128
    idx2d = x.reshape(b // RPB, RPB * h).astype(jnp.int32)
    scale_v = jnp.broadcast_to(scale.astype(jnp.float32), (L,))
    return _sc_embed(table, idx2d, scale_v, rows_per_w=b // NW, h=h, d=d)
